# baseline (speedup 1.0000x reference)
import functools

import jax
import jax.numpy as jnp
from jax import lax
from jax.experimental import pallas as pl
from jax.experimental.pallas import tpu as pltpu
from jax.experimental.pallas import tpu_sc as plsc

_N_NODES = 10000
_N_EDGES = 320000
_H = 128
_BATCH = 2

_NC = 2
_NS = 16
_L = 16
_HW = _H // 2
_NPAIR = _N_NODES // 2

_GC = 32
_SUP = 128
_NSC = _SUP // _GC
_EDGES_PER_TILE = _N_EDGES // _NS
_NSUP = 160
_EPAD = _NSUP * _SUP
_WB = 16
_WB_TOTAL = _N_NODES // _WB
_WB_PER_TILE = (_WB_TOTAL + _NS - 1) // _NS
_HS = 40
_HS_TOTAL = _NPAIR // _HS
_HS_PER_TILE = (_HS_TOTAL + _NS - 1) // _NS

_mesh = plsc.VectorSubcoreMesh(core_axis_name="c", subcore_axis_name="s")


@functools.partial(
    pl.kernel,
    out_type=jax.ShapeDtypeStruct((_BATCH, _N_NODES, _H), jnp.float32),
    mesh=_mesh,
    scratch_types=[
        pltpu.VMEM_SHARED((_N_NODES, _H), jnp.float32),
        pltpu.VMEM_SHARED((_NPAIR, _H), jnp.float32),
        pltpu.VMEM((2, _NSC, _GC), jnp.int32),
        pltpu.VMEM((2, _NSC, _GC), jnp.int32),
        pltpu.VMEM((2, _SUP), jnp.float32),
        pltpu.VMEM((_GC, _H), jnp.float32),
        pltpu.VMEM((_GC, _H), jnp.float32),
        pltpu.SemaphoreType.DMA,
        pltpu.SemaphoreType.DMA,
        pltpu.SemaphoreType.DMA,
        pltpu.SemaphoreType.DMA,
    ],
)
def _neighbor_agg(h_ref, n1_ref, n2_ref, w_ref, out_ref,
                  acc, htab, idx1s, idx2s, wvs, rb0, rb1,
                  gs0, gs1, is0, is1):
    c = lax.axis_index("c")
    s = lax.axis_index("s")
    rbufs = (rb0, rb1)
    gsems = (gs0, gs1)

    def sup_copies(k, par):
        isem = is0 if par == 0 else is1
        src = lambda ref: ref.at[c].at[s].at[k]
        return (
            pltpu.make_async_copy(src(n1_ref), idx1s.at[par], isem),
            pltpu.make_async_copy(src(n2_ref), idx2s.at[par], isem),
            pltpu.make_async_copy(src(w_ref), wvs.at[par], isem),
        )

    def sup_issue(k, par):
        for cp in sup_copies(k, par):
            cp.start()

    def sup_wait(k, par):
        for cp in sup_copies(k, par):
            cp.wait()

    def gather_issue(par, g16, l):
        pltpu.async_copy(
            htab.at[idx2s.at[par].at[g16]], rbufs[l], gsems[l])

    def gather_wait(par, g16, l):
        pltpu.make_async_copy(
            htab.at[idx2s.at[par].at[g16]], rbufs[l], gsems[l]).wait()

    for k in range(_HS_PER_TILE):
        m = s + _NS * k

        @pl.when(m < _HS_TOTAL)
        def _():
            pltpu.sync_copy(h_ref.at[c].at[pl.ds(m * _HS, _HS)],
                            htab.at[pl.ds(m * _HS, _HS)])

    zero = jnp.zeros((_L,), jnp.float32)

    def zrow(r, carry):
        for f in range(_H // _L):
            rb0[r, pl.ds(f * _L, _L)] = zero
        return carry

    lax.fori_loop(0, _WB, zrow, 0)
    for k in range(_WB_PER_TILE):
        m = s + _NS * k

        @pl.when(m < _WB_TOTAL)
        def _():
            pltpu.sync_copy(rb0.at[pl.ds(0, _WB)], acc.at[pl.ds(m * _WB, _WB)])

    plsc.subcore_barrier()

    sup_issue(0, 0)
    sup_wait(0, 0)
    gather_issue(0, 0, 0)

    def scale(par, g16, l):
        rb = rbufs[l]
        for half in range(_GC // _L):
            w16 = wvs[par, pl.ds(g16 * _GC + half * _L, _L)]
            for j16 in range(_L):
                jj = half * _L + j16
                wraw = w16[j16]
                ws = lax.abs(wraw)
                neg = wraw < 0.0
                xis = [
                    lax.bitcast_convert_type(
                        jnp.where(neg,
                                  rb[jj, pl.ds(_HW + k2 * _L, _L)],
                                  rb[jj, pl.ds(k2 * _L, _L)]),
                        jnp.int32)
                    for k2 in range(_H // 32)
                ]
                for k2, xi in enumerate(xis):
                    lo = lax.bitcast_convert_type(
                        lax.shift_left(xi, 16), jnp.float32)
                    hi = lax.bitcast_convert_type(
                        jnp.bitwise_and(xi, jnp.int32(-65536)), jnp.float32)
                    rb[jj, pl.ds(k2 * 32, _L)] = lo * ws
                    rb[jj, pl.ds(k2 * 32 + _L, _L)] = hi * ws

    def outer(ksup2, carry):
        for par in (0, 1):
            ksup = ksup2 * 2 + par
            parn = 1 - par

            @pl.when(ksup < _NSUP - 1)
            def _():
                sup_issue(ksup + 1, parn)

            def inner(j2, carry2):
                for l in range(2):
                    g16 = j2 * 2 + l
                    if l == 0:
                        gather_issue(par, g16 + 1, 1)
                    else:
                        @pl.when(j2 < _NSC // 2 - 1)
                        def _():
                            gather_issue(par, g16 + 1, 0)

                        @pl.when((j2 == _NSC // 2 - 1) & (ksup < _NSUP - 1))
                        def _():
                            sup_wait(ksup + 1, parn)
                            gather_issue(parn, 0, 0)

                    gather_wait(par, g16, l)
                    scale(par, g16, l)
                    pltpu.sync_copy(
                        rbufs[l], acc.at[idx1s.at[par].at[g16]], add=True)
                return carry2

            lax.fori_loop(0, _NSC // 2, inner, 0)
        return carry

    lax.fori_loop(0, _NSUP // 2, outer, 0)
    plsc.subcore_barrier()

    for k in range(_WB_PER_TILE):
        m = s + _NS * k

        @pl.when(m < _WB_TOTAL)
        def _():
            pltpu.sync_copy(acc.at[pl.ds(m * _WB, _WB)], rb0.at[pl.ds(0, _WB)])
            pltpu.sync_copy(rb0.at[pl.ds(0, _WB)],
                            out_ref.at[c, pl.ds(m * _WB, _WB)])


def kernel(H, edge_weights):
    n1 = edge_weights[..., 0].astype(jnp.int32)
    n2 = edge_weights[..., 1].astype(jnp.int32)
    w = edge_weights[..., 2]

    hb = H.astype(jnp.bfloat16)
    hb = hb.reshape(_BATCH, _N_NODES, _H // 32, 2, _L)
    hb = hb.transpose(0, 1, 2, 4, 3)
    hp = lax.bitcast_convert_type(hb, jnp.float32)
    hpf = hp.reshape(_BATCH, _NPAIR, _H)

    pad = _EPAD - _EDGES_PER_TILE

    def padded(x):
        x = x.reshape(_BATCH, _NS, _EDGES_PER_TILE)
        return jnp.pad(x, ((0, 0), (0, 0), (0, pad)))

    wsigned = jnp.where((n2 & 1) == 1, -w, w)
    n1c = padded(n1).reshape(_BATCH, _NS, _NSUP, _NSC, _GC)
    n2c = padded(n2 >> 1).reshape(_BATCH, _NS, _NSUP, _NSC, _GC)
    wc = padded(wsigned).reshape(_BATCH, _NS, _NSUP, _SUP)

    return _neighbor_agg(hpf, n1c, n2c, wc)


# speedup vs baseline: 1.1054x; 1.1054x over previous
import functools

import jax
import jax.numpy as jnp
from jax import lax
from jax.experimental import pallas as pl
from jax.experimental.pallas import tpu as pltpu
from jax.experimental.pallas import tpu_sc as plsc

_N_NODES = 10000
_N_EDGES = 320000
_H = 128
_BATCH = 2

_NC = 2
_NS = 16
_L = 16
_HW = _H // 2
_NPAIR = _N_NODES // 2

_GC = 32
_SUP = 128
_NSC = _SUP // _GC
_EDGES_PER_TILE = _N_EDGES // _NS
_NSUP = 160
_EPAD = _NSUP * _SUP
_WB = 16
_WB_TOTAL = _N_NODES // _WB
_WB_PER_TILE = (_WB_TOTAL + _NS - 1) // _NS
_HS = 40
_HS_TOTAL = _NPAIR // _HS
_HS_PER_TILE = (_HS_TOTAL + _NS - 1) // _NS

_mesh = plsc.VectorSubcoreMesh(core_axis_name="c", subcore_axis_name="s")


@functools.partial(
    pl.kernel,
    out_type=jax.ShapeDtypeStruct((_BATCH, _N_NODES, _H), jnp.float32),
    mesh=_mesh,
    scratch_types=[
        pltpu.VMEM_SHARED((_N_NODES, _H), jnp.float32),
        pltpu.VMEM_SHARED((_NPAIR, _H), jnp.float32),
        pltpu.VMEM((2, _NSC, _GC), jnp.int32),
        pltpu.VMEM((2, _NSC, _GC), jnp.int32),
        pltpu.VMEM((2, _SUP), jnp.float32),
        pltpu.VMEM((_GC, _H), jnp.float32),
        pltpu.VMEM((_GC, _H), jnp.float32),
        pltpu.SemaphoreType.DMA,
        pltpu.SemaphoreType.DMA,
        pltpu.SemaphoreType.DMA,
        pltpu.SemaphoreType.DMA,
    ],
)
def _neighbor_agg(h_ref, n1_ref, n2_ref, w_ref, out_ref,
                  acc, htab, idx1s, idx2s, wvs, rb0, rb1,
                  gs0, gs1, is0, is1):
    c = lax.axis_index("c")
    s = lax.axis_index("s")
    rbufs = (rb0, rb1)
    gsems = (gs0, gs1)

    def sup_copies(k, par):
        isem = is0 if par == 0 else is1
        src = lambda ref: ref.at[c].at[s].at[k]
        return (
            pltpu.make_async_copy(src(n1_ref), idx1s.at[par], isem),
            pltpu.make_async_copy(src(n2_ref), idx2s.at[par], isem),
            pltpu.make_async_copy(src(w_ref), wvs.at[par], isem),
        )

    def sup_issue(k, par):
        for cp in sup_copies(k, par):
            cp.start()

    def sup_wait(k, par):
        for cp in sup_copies(k, par):
            cp.wait()

    def gather_issue(par, g16, l):
        pltpu.async_copy(
            htab.at[idx2s.at[par].at[g16]], rbufs[l], gsems[l])

    def gather_wait(par, g16, l):
        pltpu.make_async_copy(
            htab.at[idx2s.at[par].at[g16]], rbufs[l], gsems[l]).wait()

    for k in range(_HS_PER_TILE):
        m = s + _NS * k

        @pl.when(m < _HS_TOTAL)
        def _():
            pltpu.sync_copy(h_ref.at[c].at[pl.ds(m * _HS, _HS)],
                            htab.at[pl.ds(m * _HS, _HS)])

    zero = jnp.zeros((_L,), jnp.float32)

    def zrow(r, carry):
        for f in range(_H // _L):
            rb0[r, pl.ds(f * _L, _L)] = zero
        return carry

    lax.fori_loop(0, _WB, zrow, 0)
    for k in range(_WB_PER_TILE):
        m = s + _NS * k

        @pl.when(m < _WB_TOTAL)
        def _():
            pltpu.sync_copy(rb0.at[pl.ds(0, _WB)], acc.at[pl.ds(m * _WB, _WB)])

    plsc.subcore_barrier()

    sup_issue(0, 0)
    sup_wait(0, 0)
    gather_issue(0, 0, 0)

    def scale(par, g16, l):
        rb = rbufs[l]
        for half in range(_GC // _L):
            w16 = wvs[par, pl.ds(g16 * _GC + half * _L, _L)]
            for j16 in range(_L):
                jj = half * _L + j16
                wraw = w16[j16]
                ws = lax.abs(wraw)
                off = lax.select(wraw < 0.0, _HW, 0)
                xis = [
                    lax.bitcast_convert_type(
                        rb[jj, pl.ds(off + k2 * _L, _L)], jnp.int32)
                    for k2 in range(_H // 32)
                ]
                for k2, xi in enumerate(xis):
                    lo = lax.bitcast_convert_type(
                        lax.shift_left(xi, 16), jnp.float32)
                    hi = lax.bitcast_convert_type(
                        jnp.bitwise_and(xi, jnp.int32(-65536)), jnp.float32)
                    rb[jj, pl.ds(k2 * 32, _L)] = lo * ws
                    rb[jj, pl.ds(k2 * 32 + _L, _L)] = hi * ws

    def outer(ksup2, carry):
        for par in (0, 1):
            ksup = ksup2 * 2 + par
            parn = 1 - par

            @pl.when(ksup < _NSUP - 1)
            def _():
                sup_issue(ksup + 1, parn)

            def inner(j2, carry2):
                for l in range(2):
                    g16 = j2 * 2 + l
                    if l == 0:
                        gather_issue(par, g16 + 1, 1)
                    else:
                        @pl.when(j2 < _NSC // 2 - 1)
                        def _():
                            gather_issue(par, g16 + 1, 0)

                        @pl.when((j2 == _NSC // 2 - 1) & (ksup < _NSUP - 1))
                        def _():
                            sup_wait(ksup + 1, parn)
                            gather_issue(parn, 0, 0)

                    gather_wait(par, g16, l)
                    scale(par, g16, l)
                    pltpu.sync_copy(
                        rbufs[l], acc.at[idx1s.at[par].at[g16]], add=True)
                return carry2

            lax.fori_loop(0, _NSC // 2, inner, 0)
        return carry

    lax.fori_loop(0, _NSUP // 2, outer, 0)
    plsc.subcore_barrier()

    for k in range(_WB_PER_TILE):
        m = s + _NS * k

        @pl.when(m < _WB_TOTAL)
        def _():
            pltpu.sync_copy(acc.at[pl.ds(m * _WB, _WB)], rb0.at[pl.ds(0, _WB)])
            pltpu.sync_copy(rb0.at[pl.ds(0, _WB)],
                            out_ref.at[c, pl.ds(m * _WB, _WB)])


def kernel(H, edge_weights):
    n1 = edge_weights[..., 0].astype(jnp.int32)
    n2 = edge_weights[..., 1].astype(jnp.int32)
    w = edge_weights[..., 2]

    hb = H.astype(jnp.bfloat16)
    hb = hb.reshape(_BATCH, _N_NODES, _H // 32, 2, _L)
    hb = hb.transpose(0, 1, 2, 4, 3)
    hp = lax.bitcast_convert_type(hb, jnp.float32)
    hpf = hp.reshape(_BATCH, _NPAIR, _H)

    pad = _EPAD - _EDGES_PER_TILE

    def padded(x):
        x = x.reshape(_BATCH, _NS, _EDGES_PER_TILE)
        return jnp.pad(x, ((0, 0), (0, 0), (0, pad)))

    wsigned = jnp.where((n2 & 1) == 1, -w, w)
    n1c = padded(n1).reshape(_BATCH, _NS, _NSUP, _NSC, _GC)
    n2c = padded(n2 >> 1).reshape(_BATCH, _NS, _NSUP, _NSC, _GC)
    wc = padded(wsigned).reshape(_BATCH, _NS, _NSUP, _SUP)

    return _neighbor_agg(hpf, n1c, n2c, wc)
